# baseline (speedup 1.0000x reference)
import functools

import jax
import jax.numpy as jnp
import numpy as np
from jax.experimental import pallas as pl
from jax.experimental.pallas import tpu as pltpu

N_NODES = 10000
N_EDGES = 160000
N_OFFS = 32
LATENT = 128
OUT_DIM = 3
OBSTACLE = 1
E_MAX = 8192


def _mlp_ln_res_body(nin, res_idx, *refs):
    ins = refs[:nin]
    (w1s, b1, w2, b2, w3, b3, lns, lnb) = refs[nin : nin + 8]
    out = refs[-1]
    h = b1[...]
    for i in range(nin):
        h = h + jnp.dot(ins[i][...], w1s[pl.ds(i * LATENT, LATENT), :],
                        preferred_element_type=jnp.float32)
    h = jnp.maximum(h, 0.0)
    h = jnp.maximum(jnp.dot(h, w2[...], preferred_element_type=jnp.float32) + b2[...], 0.0)
    h = jnp.dot(h, w3[...], preferred_element_type=jnp.float32) + b3[...]
    m = jnp.mean(h, axis=-1, keepdims=True)
    v = jnp.mean((h - m) * (h - m), axis=-1, keepdims=True)
    h = (h - m) * jax.lax.rsqrt(v + 1e-5) * lns[...] + lnb[...]
    out[...] = ins[res_idx][...] + h


def _vec_spec():
    return pl.BlockSpec((1, LATENT), lambda i: (0, 0))


def _mlp_ln_res(inputs, mlp, ln, blk, res_idx):
    nin = len(inputs)
    B = inputs[0].shape[0]
    (w1, b1), (w2, b2), (w3, b3) = mlp
    grid = (B // blk,)
    in_specs = (
        [pl.BlockSpec((blk, LATENT), lambda i: (i, 0))] * nin
        + [pl.BlockSpec((nin * LATENT, LATENT), lambda i: (0, 0)),
           _vec_spec(),
           pl.BlockSpec((LATENT, LATENT), lambda i: (0, 0)),
           _vec_spec(),
           pl.BlockSpec((LATENT, LATENT), lambda i: (0, 0)),
           _vec_spec(),
           _vec_spec(),
           _vec_spec()]
    )
    fn = pl.pallas_call(
        functools.partial(_mlp_ln_res_body, nin, res_idx),
        grid=grid,
        in_specs=in_specs,
        out_specs=pl.BlockSpec((blk, LATENT), lambda i: (i, 0)),
        out_shape=jax.ShapeDtypeStruct((B, LATENT), jnp.float32),
    )
    r = lambda a: a.reshape(1, LATENT)
    return fn(*inputs, w1, r(b1), w2, r(b2), w3, r(b3), r(ln[0]), r(ln[1]))


def _enc_body(*refs):
    (x, w1, b1, w2, b2, w3, b3, lns, lnb, out) = refs
    h = jnp.maximum(jnp.dot(x[...], w1[...], preferred_element_type=jnp.float32) + b1[...], 0.0)
    h = jnp.maximum(jnp.dot(h, w2[...], preferred_element_type=jnp.float32) + b2[...], 0.0)
    h = jnp.dot(h, w3[...], preferred_element_type=jnp.float32) + b3[...]
    m = jnp.mean(h, axis=-1, keepdims=True)
    v = jnp.mean((h - m) * (h - m), axis=-1, keepdims=True)
    out[...] = (h - m) * jax.lax.rsqrt(v + 1e-5) * lns[...] + lnb[...]


def _encoder(x, mlp, ln, blk):
    B, K = x.shape
    (w1, b1), (w2, b2), (w3, b3) = mlp
    kp = max(8, -(-K // 8) * 8)
    if kp != K:
        x = jnp.pad(x, ((0, 0), (0, kp - K)))
    if w1.shape[0] != kp:
        w1 = jnp.pad(w1, ((0, kp - w1.shape[0]), (0, 0)))
    grid = (B // blk,)
    in_specs = [
        pl.BlockSpec((blk, kp), lambda i: (i, 0)),
        pl.BlockSpec((kp, LATENT), lambda i: (0, 0)),
        _vec_spec(),
        pl.BlockSpec((LATENT, LATENT), lambda i: (0, 0)),
        _vec_spec(),
        pl.BlockSpec((LATENT, LATENT), lambda i: (0, 0)),
        _vec_spec(),
        _vec_spec(),
        _vec_spec(),
    ]
    fn = pl.pallas_call(
        _enc_body,
        grid=grid,
        in_specs=in_specs,
        out_specs=pl.BlockSpec((blk, LATENT), lambda i: (i, 0)),
        out_shape=jax.ShapeDtypeStruct((B, LATENT), jnp.float32),
    )
    r = lambda a: a.reshape(1, LATENT)
    return fn(x, w1, r(b1), w2, r(b2), w3, r(b3), r(ln[0]), r(ln[1]))


def _dec_body(*refs):
    (x, w1, b1, w2, b2, w3, b3, out) = refs
    h = jnp.maximum(jnp.dot(x[...], w1[...], preferred_element_type=jnp.float32) + b1[...], 0.0)
    h = jnp.maximum(jnp.dot(h, w2[...], preferred_element_type=jnp.float32) + b2[...], 0.0)
    out[...] = jnp.dot(h, w3[...], preferred_element_type=jnp.float32) + b3[...]


def _decoder(x, mlp, blk):
    B = x.shape[0]
    (w1, b1), (w2, b2), (w3, b3) = mlp
    w3p = jnp.pad(w3, ((0, 0), (0, LATENT - w3.shape[1])))
    b3p = jnp.pad(b3, ((0, LATENT - b3.shape[0]),))
    grid = (B // blk,)
    in_specs = [
        pl.BlockSpec((blk, LATENT), lambda i: (i, 0)),
        pl.BlockSpec((LATENT, LATENT), lambda i: (0, 0)),
        _vec_spec(),
        pl.BlockSpec((LATENT, LATENT), lambda i: (0, 0)),
        _vec_spec(),
        pl.BlockSpec((LATENT, LATENT), lambda i: (0, 0)),
        _vec_spec(),
    ]
    fn = pl.pallas_call(
        _dec_body,
        grid=grid,
        in_specs=in_specs,
        out_specs=pl.BlockSpec((blk, LATENT), lambda i: (i, 0)),
        out_shape=jax.ShapeDtypeStruct((B, LATENT), jnp.float32),
    )
    r = lambda a: a.reshape(1, LATENT)
    return fn(x, w1, r(b1), w2, r(b2), w3p, r(b3p))[:, :OUT_DIM]


def _world_edges(node_offs, world_pos, node_type, thresh=0.03):
    N = world_pos.shape[0]
    BLK = 500
    seg = jnp.searchsorted(node_offs, jnp.arange(N, dtype=node_offs.dtype), side="right").astype(jnp.int32) - 1
    obs = node_type == OBSTACLE
    dst_flat = jnp.tile(jnp.arange(N, dtype=jnp.int32), BLK)
    t2 = jnp.float32(thresh * thresh)

    def body(carry, i0):
        buf_s, buf_d, cnt = carry
        rows = i0 + jnp.arange(BLK, dtype=jnp.int32)
        rel = world_pos[rows][:, None, :] - world_pos[None, :, :]
        d2 = jnp.sum(rel * rel, -1)
        cond = obs[rows][:, None] & (~obs)[None, :] & (seg[rows][:, None] == seg[None, :]) & (seg[rows][:, None] >= 0) & (d2 < t2)
        flat = cond.reshape(-1)
        pos = cnt + jnp.cumsum(flat.astype(jnp.int32)) - 1
        pos = jnp.where(flat, pos, E_MAX)
        src_flat = jnp.repeat(rows, N)
        buf_s = buf_s.at[pos].set(src_flat, mode="drop")
        buf_d = buf_d.at[pos].set(dst_flat, mode="drop")
        cnt = cnt + jnp.sum(flat.astype(jnp.int32))
        return (buf_s, buf_d, cnt), None

    init = (jnp.zeros((E_MAX,), jnp.int32), jnp.zeros((E_MAX,), jnp.int32), jnp.int32(0))
    (buf_s, buf_d, cnt), _ = jax.lax.scan(body, init, jnp.arange(0, N, BLK, dtype=jnp.int32))
    valid = jnp.arange(E_MAX, dtype=jnp.int32) < cnt
    wsrcs = jnp.concatenate([buf_s, buf_d])
    wdsts = jnp.concatenate([buf_d, buf_s])
    wmask = jnp.concatenate([valid, valid])
    return wsrcs, wdsts, wmask


def kernel(node_offs, node_type, mesh_pos, world_pos, known_vel, srcs, dsts, params):
    N = mesh_pos.shape[0]
    wsrcs, wdsts, wmask = _world_edges(node_offs, world_pos, node_type)

    oh = jax.nn.one_hot(node_type, 9, dtype=jnp.float32)
    node_features = jnp.concatenate([known_vel, oh], -1)
    rel_mesh = mesh_pos[srcs] - mesh_pos[dsts]
    rel_wm = world_pos[srcs] - mesh_pos[dsts]
    mesh_ef = jnp.concatenate(
        [rel_mesh, jnp.linalg.norm(rel_mesh, axis=-1, keepdims=True),
         rel_wm, jnp.linalg.norm(rel_wm, axis=-1, keepdims=True)], -1)
    rel_w = world_pos[wsrcs] - world_pos[wdsts]
    world_ef = jnp.concatenate([rel_w, jnp.linalg.norm(rel_w, axis=-1, keepdims=True)], -1)

    def norm(nm, x):
        m, s = nm
        return (x - m) / s

    v = _encoder(norm(params["node_norm"], node_features), params["node_enc"]["mlp"],
                 params["node_enc"]["ln"], blk=2000)
    e0 = _encoder(norm(params["mesh_norm"], mesh_ef), params["mesh_enc"]["mlp"],
                  params["mesh_enc"]["ln"], blk=2000)
    e1 = _encoder(norm(params["world_norm"], world_ef), params["world_enc"]["mlp"],
                  params["world_enc"]["ln"], blk=2048)

    wmaskf = wmask[:, None].astype(jnp.float32)
    for st in params["steps"]:
        vs = v[srcs]
        vd = v[dsts]
        ne0 = _mlp_ln_res([vs, vd, e0], st["edge0"]["mlp"], st["edge0"]["ln"], blk=2000, res_idx=2)
        wvs = v[wsrcs]
        wvd = v[wdsts]
        ne1 = _mlp_ln_res([wvs, wvd, e1], st["edge1"]["mlp"], st["edge1"]["ln"], blk=2048, res_idx=2)
        agg0 = jax.ops.segment_sum(ne0, dsts, num_segments=N)
        agg1 = jax.ops.segment_sum(ne1 * wmaskf, wdsts, num_segments=N)
        v = _mlp_ln_res([v, agg0, agg1], st["node"]["mlp"], st["node"]["ln"], blk=2000, res_idx=0)
        e0, e1 = ne0, ne1

    out = _decoder(v, params["decoder"], blk=2000)
    m, s = params["out_norm"]
    return out * s + m


# speedup vs baseline: 34.5851x; 34.5851x over previous
import functools

import jax
import jax.numpy as jnp
from jax import lax
from jax.experimental import pallas as pl
from jax.experimental.pallas import tpu as pltpu
from jax.experimental.pallas import tpu_sc as plsc

N_NODES = 10000
LATENT = 128
OUT_DIM = 3
OBSTACLE = 1
E_MAX = 8192

NC = 2
NS = 16
NW = NC * NS
NP = 10240
CAPW = E_MAX // NW

_F32 = jnp.float32
_I32 = jnp.int32


def _mesh():
    return plsc.VectorSubcoreMesh(core_axis_name="c", subcore_axis_name="s",
                                  num_cores=NC, num_subcores=NS)


_SC_PARAMS = pltpu.CompilerParams(needs_layout_passes=False)


def _wid():
    return lax.axis_index("s") * NC + lax.axis_index("c")


def _range_body(*refs):
    (offs, lo_ref, hi_ref) = refs
    nr = NP // 128
    i2 = (lax.broadcasted_iota(_I32, (nr, 128), 0) * 128
          + lax.broadcasted_iota(_I32, (nr, 128), 1))
    lo = jnp.full((nr, 128), -1, _I32)
    hi = jnp.full((nr, 128), N_NODES, _I32)
    for j in range(32):
        oj = offs[j]
        lo = jnp.where(oj <= i2, jnp.maximum(lo, oj), lo)
        hi = jnp.where(oj > i2, jnp.minimum(hi, oj), hi)
    lo_ref[...] = lo
    hi_ref[...] = hi


def _seg_ranges(node_offs):
    nr = NP // 128
    fn = pl.pallas_call(
        _range_body,
        in_specs=[pl.BlockSpec(memory_space=pltpu.SMEM)],
        out_specs=[pl.BlockSpec((nr, 128), lambda: (0, 0)),
                   pl.BlockSpec((nr, 128), lambda: (0, 0))],
        out_shape=[jax.ShapeDtypeStruct((nr, 128), _I32),
                   jax.ShapeDtypeStruct((nr, 128), _I32)],
    )
    lo, hi = fn(node_offs.astype(_I32))
    return lo.reshape(NP), hi.reshape(NP)


def _we_body(type_hbm, wx_hbm, wy_hbm, wz_hbm, lo_hbm, hi_hbm,
             outs_hbm, outd_hbm, cnts_hbm,
             type_v, wx_v, wy_v, wz_v, lo_v, hi_v, bufs_v, bufd_v, cnt_v,
             mbuf_v, flag_v):
    w = _wid()
    pltpu.sync_copy(type_hbm, type_v)
    pltpu.sync_copy(wx_hbm, wx_v)
    pltpu.sync_copy(wy_hbm, wy_v)
    pltpu.sync_copy(wz_hbm, wz_v)
    pltpu.sync_copy(lo_hbm, lo_v)
    pltpu.sync_copy(hi_hbm, hi_v)

    rpw = NP // NW
    r0 = w * rpw
    t2 = jnp.float32(0.03 * 0.03)
    io1 = lax.iota(_I32, 16)

    def chunk_body(ic, cnt):
        base = r0 + ic * 16
        ty_c = type_v[pl.ds(base, 16)]
        x_c = wx_v[pl.ds(base, 16)]
        y_c = wy_v[pl.ds(base, 16)]
        z_c = wz_v[pl.ds(base, 16)]
        lo_c = lo_v[pl.ds(base, 16)]
        hi_c = hi_v[pl.ds(base, 16)]

        for l in range(16):
            r = base + l
            ty = ty_c[l]
            lo = lo_c[l]
            hi = hi_c[l]

            def scan_row(cnt, r=r, l=l, lo=lo, hi=hi, x_c=x_c, y_c=y_c, z_c=z_c):
                xr = x_c[l]
                yr = y_c[l]
                zr = z_c[l]
                c0 = lo - lax.rem(lo, 16)
                ncol = (hi - c0 + 15) // 16

                def col_body(j, cnt, r=r):
                    ci = c0 + j * 16
                    lanes = ci + io1
                    dx = wx_v[pl.ds(ci, 16)] - xr
                    dy = wy_v[pl.ds(ci, 16)] - yr
                    dz = wz_v[pl.ds(ci, 16)] - zr
                    d2 = dx * dx + dy * dy + dz * dz
                    tv = type_v[pl.ds(ci, 16)]
                    m = ((lanes >= lo) & (lanes < hi) & (tv != OBSTACLE)
                         & (d2 < t2))
                    flag_v[...] = jnp.zeros((16,), _I32)
                    plsc.store_scatter(flag_v, [jnp.where(m, 0, 8)],
                                       jnp.full((16,), 1, _I32))
                    anyflag = flag_v[...][0]

                    def append(cnt, ci=ci, m=m, r=r):
                        mbuf_v[...] = jnp.where(m, 1, 0)

                        def lane_body(l, cnt):
                            mv = plsc.load_gather(
                                mbuf_v, [jnp.full((16,), l, _I32)])[0]
                            off = jnp.minimum(cnt, CAPW)
                            tgt = jnp.full(
                                (16,), jnp.where(mv > 0, off, CAPW + 16), _I32)
                            plsc.store_scatter(
                                bufd_v, [tgt], jnp.full((16,), ci + l, _I32))
                            plsc.store_scatter(
                                bufs_v, [tgt], jnp.full((16,), r, _I32))
                            return cnt + mv

                        return lax.fori_loop(0, 16, lane_body, cnt)

                    return lax.cond(anyflag > 0, append, lambda c: c, cnt)

                return lax.fori_loop(0, ncol, col_body, cnt)

            cnt = lax.cond((ty == OBSTACLE) & (lo >= 0), scan_row,
                           lambda c: c, cnt)
        return cnt

    cnt = lax.fori_loop(0, rpw // 16, chunk_body, jnp.int32(0))
    cnt = jnp.minimum(cnt, CAPW)
    cnt_v[...] = jnp.full((16,), cnt, _I32)
    pltpu.sync_copy(bufs_v.at[pl.ds(0, CAPW)], outs_hbm.at[pl.ds(w * CAPW, CAPW)])
    pltpu.sync_copy(bufd_v.at[pl.ds(0, CAPW)], outd_hbm.at[pl.ds(w * CAPW, CAPW)])
    pltpu.sync_copy(cnt_v, cnts_hbm.at[w])


def _world_edges(node_offs, world_pos, node_type):
    wp = jnp.pad(world_pos, ((0, NP - N_NODES), (0, 0)))
    tp = jnp.pad(node_type, ((0, NP - N_NODES),))
    lo_t, hi_t = _seg_ranges(node_offs)
    fn = pl.kernel(
        _we_body,
        out_type=[jax.ShapeDtypeStruct((E_MAX,), _I32),
                  jax.ShapeDtypeStruct((E_MAX,), _I32),
                  jax.ShapeDtypeStruct((NW, 16), _I32)],
        mesh=_mesh(),
        compiler_params=_SC_PARAMS,
        scratch_types=[pltpu.VMEM((NP,), _I32),
                       pltpu.VMEM((NP,), _F32),
                       pltpu.VMEM((NP,), _F32),
                       pltpu.VMEM((NP,), _F32),
                       pltpu.VMEM((NP,), _I32),
                       pltpu.VMEM((NP,), _I32),
                       pltpu.VMEM((CAPW + 32,), _I32),
                       pltpu.VMEM((CAPW + 32,), _I32),
                       pltpu.VMEM((16,), _I32),
                       pltpu.VMEM((16,), _I32),
                       pltpu.VMEM((16,), _I32)],
    )
    outs, outd, cnts = fn(tp, wp[:, 0], wp[:, 1], wp[:, 2], lo_t, hi_t)
    cnts = cnts[:, 0]
    slot = jnp.arange(E_MAX, dtype=_I32)
    valid = (slot % CAPW) < cnts[slot // CAPW]
    buf_s = jnp.where(valid, outs, 0)
    buf_d = jnp.where(valid, outd, 0)
    wsrcs = jnp.concatenate([buf_s, buf_d])
    wdsts = jnp.concatenate([buf_d, buf_s])
    wmask = jnp.concatenate([valid, valid])
    return wsrcs, wdsts, wmask


def _gather_body(C, D, NB, table_hbm, idx_hbm, out_hbm, idx_v, rows_v, gsem, wsem):
    w = _wid()
    ncmax = -(-C // NW)
    ngrp = -(-ncmax // NB)

    def grp(g, _):
        cs = []
        for b in range(NB):
            j = g * NB + b
            c = jnp.where(w + j * NW < C, w + j * NW, w)
            cs.append(c)
            pltpu.sync_copy(idx_hbm.at[c], idx_v.at[b])
        hs = [pltpu.async_copy(table_hbm.at[idx_v.at[b]], rows_v.at[b], gsem)
              for b in range(NB)]
        for h in hs:
            h.wait()
        ws = [pltpu.async_copy(rows_v.at[b],
                               out_hbm.at[pl.ds(cs[b] * 128, 128)], wsem)
              for b in range(NB)]
        for h in ws:
            h.wait()
        return 0

    lax.fori_loop(0, ngrp, grp, 0)


def _sc_gather(table, idx, nbuf=4):
    NT, D = table.shape
    E = idx.shape[0]
    C = E // 128
    idx2 = idx.reshape(C, 128)
    fn = pl.kernel(
        functools.partial(_gather_body, C, D, nbuf),
        out_type=jax.ShapeDtypeStruct((E, D), _F32),
        mesh=_mesh(),
        compiler_params=_SC_PARAMS,
        scratch_types=[pltpu.VMEM((nbuf, 128), _I32),
                       pltpu.VMEM((nbuf, 128, D), _F32),
                       pltpu.SemaphoreType.DMA,
                       pltpu.SemaphoreType.DMA],
    )
    return fn(table, idx2)


def _segsum_body(C, zeros_hbm, data_hbm, idx_hbm, out_hbm,
                 idx_v, data_v, acc_sh, sem):
    w = _wid()
    cid = lax.axis_index("c")
    sid = lax.axis_index("s")
    pt = NP // NS
    pltpu.sync_copy(zeros_hbm, acc_sh.at[pl.ds(sid * pt, pt)])
    plsc.subcore_barrier()

    nc = -(-C // NW)

    def body(i, _):
        j = w + i * NW

        @pl.when(j < C)
        def _():
            pltpu.sync_copy(idx_hbm.at[j], idx_v)
            pltpu.async_copy(data_hbm.at[pl.ds(j * 128, 128)], data_v, sem).wait()
            pltpu.sync_copy(data_v, acc_sh.at[idx_v], add=True)

        return 0

    lax.fori_loop(0, nc, body, 0)
    plsc.subcore_barrier()
    pltpu.sync_copy(acc_sh.at[pl.ds(sid * pt, pt)],
                    out_hbm.at[cid, pl.ds(sid * pt, pt)])


def _sc_segsum(data, idx):
    E = data.shape[0]
    C = E // 128
    idx2 = idx.reshape(C, 128)
    zeros = jnp.zeros((NP // NS, LATENT), _F32)
    fn = pl.kernel(
        functools.partial(_segsum_body, C),
        out_type=jax.ShapeDtypeStruct((NC, NP, LATENT), _F32),
        mesh=_mesh(),
        compiler_params=_SC_PARAMS,
        scratch_types=[pltpu.VMEM((128,), _I32),
                       pltpu.VMEM((128, LATENT), _F32),
                       pltpu.VMEM_SHARED((NP, LATENT), _F32),
                       pltpu.SemaphoreType.DMA],
    )
    return fn(zeros, data, idx2)


def _vec_spec():
    return pl.BlockSpec((1, LATENT), lambda i: (0, 0))


def _w_specs(nin):
    return [pl.BlockSpec((nin * LATENT, LATENT), lambda i: (0, 0)),
            _vec_spec(),
            pl.BlockSpec((LATENT, LATENT), lambda i: (0, 0)),
            _vec_spec(),
            pl.BlockSpec((LATENT, LATENT), lambda i: (0, 0)),
            _vec_spec(),
            _vec_spec(),
            _vec_spec()]


def _w_args(mlp, ln):
    (w1, b1), (w2, b2), (w3, b3) = mlp
    r = lambda a: a.reshape(1, LATENT)
    return [w1, r(b1), w2, r(b2), w3, r(b3), r(ln[0]), r(ln[1])]


def _ln_tail(h, lns, lnb):
    m = jnp.mean(h, axis=-1, keepdims=True)
    v = jnp.mean((h - m) * (h - m), axis=-1, keepdims=True)
    return (h - m) / jnp.sqrt(v + 1e-5) * lns[...] + lnb[...]


def _mlp23(h, w2, b2, w3, b3, prec=None):
    h = jnp.maximum(jnp.dot(h, w2[...], preferred_element_type=_F32, precision=prec) + b2[...], 0.0)
    return jnp.dot(h, w3[...], preferred_element_type=_F32, precision=prec) + b3[...]


def _edge_body(*refs):
    (gs, gd, e, w1s, b1, w2, b2, w3, b3, lns, lnb, out) = refs
    x = jnp.concatenate([gs[...], gd[...], e[...]], axis=-1)
    h = jnp.dot(x, w1s[...], preferred_element_type=_F32) + b1[...]
    h = _mlp23(jnp.maximum(h, 0.0), w2, b2, w3, b3)
    out[...] = e[...] + _ln_tail(h, lns, lnb)


def _edge_mlp(g, e, st, blk):
    E = e.shape[0]
    nb = E // blk
    grid = (nb,)
    in_specs = ([pl.BlockSpec((blk, LATENT), lambda i: (i, 0)),
                 pl.BlockSpec((blk, LATENT), lambda i, o=nb: (i + o, 0)),
                 pl.BlockSpec((blk, LATENT), lambda i: (i, 0))]
                + _w_specs(3))
    fn = pl.pallas_call(
        _edge_body,
        grid=grid,
        in_specs=in_specs,
        out_specs=pl.BlockSpec((blk, LATENT), lambda i: (i, 0)),
        out_shape=jax.ShapeDtypeStruct((E, LATENT), _F32),
    )
    return fn(g, g, e, *_w_args(st["mlp"], st["ln"]))


def _node_body(*refs):
    (v, a0a, a0b, a1a, a1b, w1s, b1, w2, b2, w3, b3, lns, lnb, out) = refs
    x = jnp.concatenate([v[...], a0a[0] + a0b[0], a1a[0] + a1b[0]], axis=-1)
    h = jnp.dot(x, w1s[...], preferred_element_type=_F32) + b1[...]
    h = _mlp23(jnp.maximum(h, 0.0), w2, b2, w3, b3)
    out[...] = v[...] + _ln_tail(h, lns, lnb)


def _node_mlp(v, agg0, agg1, st, blk):
    N = v.shape[0]
    grid = (N // blk,)
    p3 = lambda r: pl.BlockSpec((1, blk, LATENT), lambda i, rr=r: (rr, i, 0))
    in_specs = ([pl.BlockSpec((blk, LATENT), lambda i: (i, 0)),
                 p3(0), p3(1), p3(0), p3(1)]
                + _w_specs(3))
    fn = pl.pallas_call(
        _node_body,
        grid=grid,
        in_specs=in_specs,
        out_specs=pl.BlockSpec((blk, LATENT), lambda i: (i, 0)),
        out_shape=jax.ShapeDtypeStruct((N, LATENT), _F32),
    )
    return fn(v, agg0, agg0, agg1, agg1, *_w_args(st["mlp"], st["ln"]))


def _mesh_enc_body(*refs):
    (ps, pd, nm, ns_, w1, b1, w2, b2, w3, b3, lns, lnb, out) = refs
    rel_m = ps[:, 0:3] - pd[:, 0:3]
    rel_w = ps[:, 3:6] - pd[:, 0:3]
    n_m = jnp.sqrt(jnp.sum(rel_m * rel_m, axis=-1, keepdims=True))
    n_w = jnp.sqrt(jnp.sum(rel_w * rel_w, axis=-1, keepdims=True))
    x = jnp.concatenate([rel_m, n_m, rel_w, n_w], axis=-1)
    x = (x - nm[...]) / ns_[...]
    h = b1[...] + jnp.zeros_like(out[...])
    for kk in range(8):
        h = h + x[:, kk:kk + 1] * w1[pl.ds(kk, 1), :]
    h = _mlp23(jnp.maximum(h, 0.0), w2, b2, w3, b3, lax.Precision.HIGHEST)
    out[...] = _ln_tail(h, lns, lnb)


def _world_enc_body(*refs):
    (ps, pd, nm, ns_, w1, b1, w2, b2, w3, b3, lns, lnb, out) = refs
    rel = ps[:, 3:6] - pd[:, 3:6]
    n = jnp.sqrt(jnp.sum(rel * rel, axis=-1, keepdims=True))
    x = jnp.concatenate([rel, n], axis=-1)
    x = (x - nm[...]) / ns_[...]
    h = b1[...] + jnp.zeros_like(out[...])
    for kk in range(4):
        h = h + x[:, kk:kk + 1] * w1[pl.ds(kk, 1), :]
    h = _mlp23(jnp.maximum(h, 0.0), w2, b2, w3, b3, lax.Precision.HIGHEST)
    out[...] = _ln_tail(h, lns, lnb)


def _pos_encoder(body, gp, kdim, mlp, ln, norm, blk):
    E = gp.shape[0] // 2
    nb = E // blk
    (w1, b1), (w2, b2), (w3, b3) = mlp
    m, s = norm
    grid = (nb,)
    kspec = pl.BlockSpec((1, kdim), lambda i: (0, 0))
    in_specs = ([pl.BlockSpec((blk, LATENT), lambda i: (i, 0)),
                 pl.BlockSpec((blk, LATENT), lambda i, o=nb: (i + o, 0)),
                 kspec,
                 kspec,
                 pl.BlockSpec((kdim, LATENT), lambda i: (0, 0)),
                 _vec_spec(),
                 pl.BlockSpec((LATENT, LATENT), lambda i: (0, 0)),
                 _vec_spec(),
                 pl.BlockSpec((LATENT, LATENT), lambda i: (0, 0)),
                 _vec_spec(),
                 _vec_spec(),
                 _vec_spec()])
    fn = pl.pallas_call(
        body,
        grid=grid,
        in_specs=in_specs,
        out_specs=pl.BlockSpec((blk, LATENT), lambda i: (i, 0)),
        out_shape=jax.ShapeDtypeStruct((E, LATENT), _F32),
    )
    r = lambda a: a.reshape(1, LATENT)
    return fn(gp, gp, m.reshape(1, kdim), s.reshape(1, kdim),
              w1, r(b1), w2, r(b2), w3, r(b3), r(ln[0]), r(ln[1]))


def _enc_body(kp, *refs):
    (x, w1, b1, w2, b2, w3, b3, lns, lnb, out) = refs
    h = jnp.dot(x[...], w1[...], preferred_element_type=_F32) + b1[...]
    h = _mlp23(jnp.maximum(h, 0.0), w2, b2, w3, b3)
    out[...] = _ln_tail(h, lns, lnb)


def _node_encoder(x, mlp, ln, norm, blk):
    B, K = x.shape
    (w1, b1), (w2, b2), (w3, b3) = mlp
    m, s = norm
    x = (x - m) / s
    w1f = w1
    kp = -(-K // 8) * 8
    if kp != K:
        x = jnp.pad(x, ((0, 0), (0, kp - K)))
        w1f = jnp.pad(w1f, ((0, kp - K), (0, 0)))
    grid = (B // blk,)
    in_specs = [pl.BlockSpec((blk, kp), lambda i: (i, 0)),
                pl.BlockSpec((kp, LATENT), lambda i: (0, 0)),
                _vec_spec(),
                pl.BlockSpec((LATENT, LATENT), lambda i: (0, 0)),
                _vec_spec(),
                pl.BlockSpec((LATENT, LATENT), lambda i: (0, 0)),
                _vec_spec(),
                _vec_spec(),
                _vec_spec()]
    fn = pl.pallas_call(
        functools.partial(_enc_body, kp),
        grid=grid,
        in_specs=in_specs,
        out_specs=pl.BlockSpec((blk, LATENT), lambda i: (i, 0)),
        out_shape=jax.ShapeDtypeStruct((B, LATENT), _F32),
    )
    r = lambda a: a.reshape(1, LATENT)
    return fn(x, w1f, r(b1), w2, r(b2), w3, r(b3), r(ln[0]), r(ln[1]))


def _dec_body(*refs):
    (x, w1, b1, w2, b2, w3, b3, out) = refs
    h = jnp.maximum(jnp.dot(x[...], w1[...], preferred_element_type=_F32, precision=lax.Precision.HIGHEST) + b1[...], 0.0)
    out[...] = _mlp23(h, w2, b2, w3, b3, lax.Precision.HIGHEST)


def _decoder(x, mlp, out_norm, blk):
    B = x.shape[0]
    (w1, b1), (w2, b2), (w3, b3) = mlp
    m, s = out_norm
    w3f = jnp.pad(w3, ((0, 0), (0, LATENT - w3.shape[1])))
    b3f = jnp.pad(b3, ((0, LATENT - b3.shape[0]),))
    grid = (B // blk,)
    in_specs = [pl.BlockSpec((blk, LATENT), lambda i: (i, 0)),
                pl.BlockSpec((LATENT, LATENT), lambda i: (0, 0)),
                _vec_spec(),
                pl.BlockSpec((LATENT, LATENT), lambda i: (0, 0)),
                _vec_spec(),
                pl.BlockSpec((LATENT, LATENT), lambda i: (0, 0)),
                _vec_spec()]
    fn = pl.pallas_call(
        _dec_body,
        grid=grid,
        in_specs=in_specs,
        out_specs=pl.BlockSpec((blk, LATENT), lambda i: (i, 0)),
        out_shape=jax.ShapeDtypeStruct((B, LATENT), _F32),
    )
    r = lambda a: a.reshape(1, LATENT)
    return fn(x, w1, r(b1), w2, r(b2), w3f, r(b3f))[:, :OUT_DIM] * s + m


def kernel(node_offs, node_type, mesh_pos, world_pos, known_vel, srcs, dsts, params):
    wsrcs, wdsts, wmask = _world_edges(node_offs, world_pos, node_type)

    ptab = jnp.concatenate(
        [mesh_pos, world_pos, jnp.zeros((N_NODES, 122), _F32)], axis=1)
    sd_idx = jnp.concatenate([srcs, dsts]).astype(_I32)
    w_idx = jnp.concatenate([wsrcs, wdsts]).astype(_I32)
    gp = _sc_gather(ptab, sd_idx)
    gpw = _sc_gather(ptab, w_idx)

    oh = jax.nn.one_hot(node_type, 9, dtype=_F32)
    node_features = jnp.concatenate([known_vel, oh], -1)
    v = _node_encoder(node_features, params["node_enc"]["mlp"],
                      params["node_enc"]["ln"], params["node_norm"], blk=2000)
    e0 = _pos_encoder(_mesh_enc_body, gp, 8, params["mesh_enc"]["mlp"],
                      params["mesh_enc"]["ln"], params["mesh_norm"], blk=2000)
    e1 = _pos_encoder(_world_enc_body, gpw, 4, params["world_enc"]["mlp"],
                      params["world_enc"]["ln"], params["world_norm"], blk=2048)

    wdst_eff = jnp.where(wmask, wdsts, N_NODES).astype(_I32)
    dsts_i = dsts.astype(_I32)
    for st in params["steps"]:
        gv = _sc_gather(v, sd_idx)
        gvw = _sc_gather(v, w_idx)
        ne0 = _edge_mlp(gv, e0, st["edge0"], blk=2000)
        ne1 = _edge_mlp(gvw, e1, st["edge1"], blk=2048)
        agg0 = _sc_segsum(ne0, dsts_i)
        agg1 = _sc_segsum(ne1, wdst_eff)
        v = _node_mlp(v, agg0, agg1, st["node"], blk=2000)
        e0, e1 = ne0, ne1

    return _decoder(v, params["decoder"], params["out_norm"], blk=2000)
